# Initial kernel scaffold; baseline (speedup 1.0000x reference)
import jax
import jax.numpy as jnp
from jax.experimental import pallas as pl


def kernel(score, target):
    raise NotImplementedError("write your pallas kernel here")


# speedup vs baseline: 14.8725x; 14.8725x over previous
import functools

import jax
import jax.numpy as jnp
from jax import lax
from jax.experimental import pallas as pl
from jax.experimental.pallas import tpu as pltpu
from jax.experimental.pallas import tpu_sc as plsc

THRESH = 0.2
MIN_KEPT_FRAC = 0.01

N = 8 * 512 * 512
ROWS, COLS = 2048, 1024
BLK_ROWS = 256

HROWS, HCOLS = 264, 128
H = HROWS * HCOLS
NBUCKETS = 32768
TRASH = NBUCKETS

NTILES = 32
PER_TILE = N // NTILES
CHUNK = 4096

K_KEPT = int(MIN_KEPT_FRAC * N)


def _elemwise_body(s_ref, t_ref, id_ref, loss_ref):
    s = s_ref[...]
    t = t_ref[...]
    a = jnp.abs(s)
    loss = jnp.maximum(s, 0.0) - s * t + jnp.log1p(jnp.exp(-a))
    pred = jnp.abs(jax.nn.sigmoid(s) - 0.5)
    mask = pred < THRESH
    bits = lax.bitcast_convert_type(a, jnp.int32)
    bucket = jnp.minimum(lax.shift_right_logical(bits, 15), NBUCKETS - 1)
    id_ref[...] = jnp.where(mask, bucket, TRASH)
    loss_ref[...] = loss


def _hist_body(ids_hbm, loss_hbm, cnt_out, sum_out, ids_v, loss_v, cnt_h, sum_h):
    c = lax.axis_index("c")
    s = lax.axis_index("s")
    wid = s * 2 + c
    base = wid * PER_TILE

    zeros16 = jnp.zeros((16,), jnp.float32)
    ones16 = jnp.ones((16,), jnp.float32)

    @pl.loop(0, H // 16)
    def _zero(i):
        cnt_h[pl.ds(i * 16, 16)] = zeros16
        sum_h[pl.ds(i * 16, 16)] = zeros16

    @pl.loop(0, PER_TILE // CHUNK)
    def _chunk(ci):
        off = base + ci * CHUNK
        pltpu.sync_copy(ids_hbm.at[pl.ds(off, CHUNK)], ids_v)
        pltpu.sync_copy(loss_hbm.at[pl.ds(off, CHUNK)], loss_v)

        @pl.loop(0, CHUNK // 16)
        def _vec(j):
            idx = ids_v[pl.ds(j * 16, 16)]
            x = loss_v[pl.ds(j * 16, 16)]
            plsc.addupdate_scatter(cnt_h, [idx], ones16)
            plsc.addupdate_scatter(sum_h, [idx], x)

    pltpu.sync_copy(cnt_h, cnt_out.at[wid])
    pltpu.sync_copy(sum_h, sum_out.at[wid])


def _masked_roll_add(x, sh, axis, pos):
    return x + jnp.where(pos >= sh, pltpu.roll(x, sh, axis), 0.0)


def _select_body(cnt_ref, sum_ref, out_ref, acc_c, acc_s):
    step = pl.program_id(0)

    @pl.when(step == 0)
    def _init():
        acc_c[...] = jnp.zeros_like(acc_c)
        acc_s[...] = jnp.zeros_like(acc_s)

    acc_c[...] += cnt_ref[0]
    acc_s[...] += sum_ref[0]

    @pl.when(step == NTILES - 1)
    def _finish():
        lane = lax.broadcasted_iota(jnp.int32, (HROWS, HCOLS), 1)
        row = lax.broadcasted_iota(jnp.int32, (HROWS, HCOLS), 0)
        valid = (row * HCOLS + lane) < NBUCKETS
        cnt = jnp.where(valid, acc_c[...], 0.0)
        lsum = jnp.where(valid, acc_s[...], 0.0)

        x = cnt
        for sh in (1, 2, 4, 8, 16, 32, 64):
            x = _masked_roll_add(x, sh, 1, lane)
        rowt = jnp.broadcast_to(x[:, HCOLS - 1:HCOLS], (HROWS, HCOLS))
        z = rowt
        for sh in (1, 2, 4, 8, 16, 32, 64, 128, 256):
            z = _masked_roll_add(z, sh, 0, row)
        excl = (x + (z - rowt)) - cnt

        total = jnp.sum(cnt)
        k = jnp.minimum(jnp.float32(K_KEPT), total)
        take = jnp.clip(k - excl, 0.0, cnt)
        num = jnp.sum(lsum * take / jnp.maximum(cnt, 1.0))
        out_ref[...] = jnp.reshape(num / jnp.maximum(k, 1.0), (1, 1))


def _make_elemwise():
    return pl.pallas_call(
        _elemwise_body,
        grid=(ROWS // BLK_ROWS,),
        in_specs=[
            pl.BlockSpec((BLK_ROWS, COLS), lambda i: (i, 0)),
            pl.BlockSpec((BLK_ROWS, COLS), lambda i: (i, 0)),
        ],
        out_specs=[
            pl.BlockSpec((BLK_ROWS, COLS), lambda i: (i, 0)),
            pl.BlockSpec((BLK_ROWS, COLS), lambda i: (i, 0)),
        ],
        out_shape=[
            jax.ShapeDtypeStruct((ROWS, COLS), jnp.int32),
            jax.ShapeDtypeStruct((ROWS, COLS), jnp.float32),
        ],
    )


def _make_hist():
    mesh = plsc.VectorSubcoreMesh(core_axis_name="c", subcore_axis_name="s")
    return pl.kernel(
        _hist_body,
        out_type=(
            jax.ShapeDtypeStruct((NTILES, H), jnp.float32),
            jax.ShapeDtypeStruct((NTILES, H), jnp.float32),
        ),
        mesh=mesh,
        scratch_types=[
            pltpu.VMEM((CHUNK,), jnp.int32),
            pltpu.VMEM((CHUNK,), jnp.float32),
            pltpu.VMEM((H,), jnp.float32),
            pltpu.VMEM((H,), jnp.float32),
        ],
        compiler_params=pltpu.CompilerParams(needs_layout_passes=False),
    )


def _make_select():
    return pl.pallas_call(
        _select_body,
        grid=(NTILES,),
        in_specs=[
            pl.BlockSpec((1, HROWS, HCOLS), lambda i: (i, 0, 0)),
            pl.BlockSpec((1, HROWS, HCOLS), lambda i: (i, 0, 0)),
        ],
        out_specs=pl.BlockSpec((1, 1), lambda i: (0, 0)),
        out_shape=jax.ShapeDtypeStruct((1, 1), jnp.float32),
        scratch_shapes=[
            pltpu.VMEM((HROWS, HCOLS), jnp.float32),
            pltpu.VMEM((HROWS, HCOLS), jnp.float32),
        ],
    )


def kernel(score, target):
    s = score.reshape(ROWS, COLS)
    t = target.reshape(ROWS, COLS)
    ids, losses = _make_elemwise()(s, t)
    cnt_h, sum_h = _make_hist()(ids.reshape(N), losses.reshape(N))
    out = _make_select()(
        cnt_h.reshape(NTILES, HROWS, HCOLS),
        sum_h.reshape(NTILES, HROWS, HCOLS),
    )
    return out.reshape(())


# speedup vs baseline: 16.9186x; 1.1376x over previous
import functools

import jax
import jax.numpy as jnp
from jax import lax
from jax.experimental import pallas as pl
from jax.experimental.pallas import tpu as pltpu
from jax.experimental.pallas import tpu_sc as plsc

THRESH = 0.2
MIN_KEPT_FRAC = 0.01
ABS_THRESH = 0.8472978603872037

N = 8 * 512 * 512
ROWS, COLS = 2048, 1024
BLK_ROWS = 256

HROWS, HCOLS = 264, 128
H = HROWS * HCOLS
NBUCKETS = 32768
TRASH = NBUCKETS

NTILES = 32
PER_TILE = N // NTILES
CHUNK = 4096

K_KEPT = int(MIN_KEPT_FRAC * N)


def _elemwise_body(s_ref, t_ref, id_ref, loss_ref):
    s = s_ref[...]
    t = t_ref[...]
    a = jnp.abs(s)
    loss = jnp.maximum(s, 0.0) - s * t + jnp.log1p(jnp.exp(-a))
    mask = a < ABS_THRESH
    bits = lax.bitcast_convert_type(a, jnp.int32)
    bucket = jnp.minimum(lax.shift_right_logical(bits, 15), NBUCKETS - 1)
    id_ref[...] = jnp.where(mask, bucket, TRASH)
    loss_ref[...] = loss


NCHUNK = PER_TILE // CHUNK


def _hist_body(ids_hbm, loss_hbm, cnt_out, sum_out,
               ids_v0, loss_v0, ids_v1, loss_v1, cnt_h, sum_h, sem0, sem1):
    c = lax.axis_index("c")
    s = lax.axis_index("s")
    wid = s * 2 + c
    base = wid * PER_TILE

    zeros16 = jnp.zeros((16,), jnp.float32)
    ones16 = jnp.ones((16,), jnp.float32)

    ids_bufs = (ids_v0, ids_v1)
    loss_bufs = (loss_v0, loss_v1)
    sems = (sem0, sem1)

    def issue(ci, b):
        off = base + ci * CHUNK
        pltpu.async_copy(ids_hbm.at[pl.ds(off, CHUNK)], ids_bufs[b], sems[b])
        pltpu.async_copy(loss_hbm.at[pl.ds(off, CHUNK)], loss_bufs[b], sems[b])

    def drain(ci, b):
        off = base + ci * CHUNK
        pltpu.make_async_copy(ids_hbm.at[pl.ds(off, CHUNK)], ids_bufs[b], sems[b]).wait()
        pltpu.make_async_copy(loss_hbm.at[pl.ds(off, CHUNK)], loss_bufs[b], sems[b]).wait()

    def process(b):
        ids_v = ids_bufs[b]
        loss_v = loss_bufs[b]

        @pl.loop(0, CHUNK // 64)
        def _vec(j):
            for u in range(4):
                o = (j * 4 + u) * 16
                idx = ids_v[pl.ds(o, 16)]
                x = loss_v[pl.ds(o, 16)]
                plsc.addupdate_scatter(cnt_h, [idx], ones16)
                plsc.addupdate_scatter(sum_h, [idx], x)

    issue(0, 0)
    issue(1, 1)

    @pl.loop(0, H // 16)
    def _zero(i):
        cnt_h[pl.ds(i * 16, 16)] = zeros16
        sum_h[pl.ds(i * 16, 16)] = zeros16

    @pl.loop(0, NCHUNK - 2, step=2)
    def _outer(ci):
        for b in range(2):
            drain(ci + b, b)
            process(b)
            issue(ci + b + 2, b)

    drain(NCHUNK - 2, 0)
    process(0)
    drain(NCHUNK - 1, 1)
    process(1)

    pltpu.sync_copy(cnt_h, cnt_out.at[wid])
    pltpu.sync_copy(sum_h, sum_out.at[wid])


def _masked_roll_add(x, sh, axis, pos):
    return x + jnp.where(pos >= sh, pltpu.roll(x, sh, axis), 0.0)


def _select_body(cnt_ref, sum_ref, out_ref, acc_c, acc_s):
    step = pl.program_id(0)

    @pl.when(step == 0)
    def _init():
        acc_c[...] = jnp.zeros_like(acc_c)
        acc_s[...] = jnp.zeros_like(acc_s)

    acc_c[...] += cnt_ref[0]
    acc_s[...] += sum_ref[0]

    @pl.when(step == NTILES - 1)
    def _finish():
        lane = lax.broadcasted_iota(jnp.int32, (HROWS, HCOLS), 1)
        row = lax.broadcasted_iota(jnp.int32, (HROWS, HCOLS), 0)
        valid = (row * HCOLS + lane) < NBUCKETS
        cnt = jnp.where(valid, acc_c[...], 0.0)
        lsum = jnp.where(valid, acc_s[...], 0.0)

        x = cnt
        for sh in (1, 2, 4, 8, 16, 32, 64):
            x = _masked_roll_add(x, sh, 1, lane)
        rowt = jnp.broadcast_to(x[:, HCOLS - 1:HCOLS], (HROWS, HCOLS))
        z = rowt
        for sh in (1, 2, 4, 8, 16, 32, 64, 128, 256):
            z = _masked_roll_add(z, sh, 0, row)
        excl = (x + (z - rowt)) - cnt

        total = jnp.sum(cnt)
        k = jnp.minimum(jnp.float32(K_KEPT), total)
        take = jnp.clip(k - excl, 0.0, cnt)
        num = jnp.sum(lsum * take / jnp.maximum(cnt, 1.0))
        out_ref[...] = jnp.reshape(num / jnp.maximum(k, 1.0), (1, 1))


def _make_elemwise():
    return pl.pallas_call(
        _elemwise_body,
        grid=(ROWS // BLK_ROWS,),
        in_specs=[
            pl.BlockSpec((BLK_ROWS, COLS), lambda i: (i, 0)),
            pl.BlockSpec((BLK_ROWS, COLS), lambda i: (i, 0)),
        ],
        out_specs=[
            pl.BlockSpec((BLK_ROWS, COLS), lambda i: (i, 0)),
            pl.BlockSpec((BLK_ROWS, COLS), lambda i: (i, 0)),
        ],
        out_shape=[
            jax.ShapeDtypeStruct((ROWS, COLS), jnp.int32),
            jax.ShapeDtypeStruct((ROWS, COLS), jnp.float32),
        ],
    )


def _make_hist():
    mesh = plsc.VectorSubcoreMesh(core_axis_name="c", subcore_axis_name="s")
    return pl.kernel(
        _hist_body,
        out_type=(
            jax.ShapeDtypeStruct((NTILES, H), jnp.float32),
            jax.ShapeDtypeStruct((NTILES, H), jnp.float32),
        ),
        mesh=mesh,
        scratch_types=[
            pltpu.VMEM((CHUNK,), jnp.int32),
            pltpu.VMEM((CHUNK,), jnp.float32),
            pltpu.VMEM((CHUNK,), jnp.int32),
            pltpu.VMEM((CHUNK,), jnp.float32),
            pltpu.VMEM((H,), jnp.float32),
            pltpu.VMEM((H,), jnp.float32),
            pltpu.SemaphoreType.DMA,
            pltpu.SemaphoreType.DMA,
        ],
        compiler_params=pltpu.CompilerParams(needs_layout_passes=False),
    )


def _make_select():
    return pl.pallas_call(
        _select_body,
        grid=(NTILES,),
        in_specs=[
            pl.BlockSpec((1, HROWS, HCOLS), lambda i: (i, 0, 0)),
            pl.BlockSpec((1, HROWS, HCOLS), lambda i: (i, 0, 0)),
        ],
        out_specs=pl.BlockSpec((1, 1), lambda i: (0, 0)),
        out_shape=jax.ShapeDtypeStruct((1, 1), jnp.float32),
        scratch_shapes=[
            pltpu.VMEM((HROWS, HCOLS), jnp.float32),
            pltpu.VMEM((HROWS, HCOLS), jnp.float32),
        ],
    )


def kernel(score, target):
    s = score.reshape(ROWS, COLS)
    t = target.reshape(ROWS, COLS)
    ids, losses = _make_elemwise()(s, t)
    cnt_h, sum_h = _make_hist()(ids.reshape(N), losses.reshape(N))
    out = _make_select()(
        cnt_h.reshape(NTILES, HROWS, HCOLS),
        sum_h.reshape(NTILES, HROWS, HCOLS),
    )
    return out.reshape(())


# speedup vs baseline: 16.9346x; 1.0009x over previous
import functools

import jax
import jax.numpy as jnp
from jax import lax
from jax.experimental import pallas as pl
from jax.experimental.pallas import tpu as pltpu
from jax.experimental.pallas import tpu_sc as plsc

THRESH = 0.2
MIN_KEPT_FRAC = 0.01
ABS_THRESH = 0.8472978603872037

N = 8 * 512 * 512
ROWS, COLS = 2048, 1024
BLK_ROWS = 256

HROWS, HCOLS = 264, 128
H = HROWS * HCOLS
NBUCKETS = 32768
TRASH = NBUCKETS

NTILES = 32
PER_TILE = N // NTILES
CHUNK = 4096

K_KEPT = int(MIN_KEPT_FRAC * N)


_SOFTPLUS_COEFFS = (
    0.6931471710278221,
    -0.4999994142888598,
    0.12499137715952302,
    5.171521911596488e-05,
    -0.005358852851406005,
    0.0002188750327285275,
    0.000211068980896383,
)


def _elemwise_body(s_ref, t_ref, id_ref, loss_ref):
    s = s_ref[...]
    t = t_ref[...]
    a = jnp.abs(s)
    sp = jnp.float32(_SOFTPLUS_COEFFS[-1])
    for coef in _SOFTPLUS_COEFFS[-2::-1]:
        sp = sp * a + jnp.float32(coef)
    loss = jnp.maximum(s, 0.0) - s * t + sp
    mask = a < ABS_THRESH
    bits = lax.bitcast_convert_type(a, jnp.int32)
    bucket = jnp.minimum(lax.shift_right_logical(bits, 15), NBUCKETS - 1)
    id_ref[...] = jnp.where(mask, bucket, TRASH)
    loss_ref[...] = loss


NCHUNK = PER_TILE // CHUNK


def _hist_body(ids_hbm, loss_hbm, cnt_out, sum_out,
               ids_v0, loss_v0, ids_v1, loss_v1, cnt_h, sum_h, sem0, sem1):
    c = lax.axis_index("c")
    s = lax.axis_index("s")
    wid = s * 2 + c
    base = wid * PER_TILE

    zeros16 = jnp.zeros((16,), jnp.float32)
    ones16 = jnp.ones((16,), jnp.float32)

    ids_bufs = (ids_v0, ids_v1)
    loss_bufs = (loss_v0, loss_v1)
    sems = (sem0, sem1)

    def issue(ci, b):
        off = base + ci * CHUNK
        pltpu.async_copy(ids_hbm.at[pl.ds(off, CHUNK)], ids_bufs[b], sems[b])
        pltpu.async_copy(loss_hbm.at[pl.ds(off, CHUNK)], loss_bufs[b], sems[b])

    def drain(ci, b):
        off = base + ci * CHUNK
        pltpu.make_async_copy(ids_hbm.at[pl.ds(off, CHUNK)], ids_bufs[b], sems[b]).wait()
        pltpu.make_async_copy(loss_hbm.at[pl.ds(off, CHUNK)], loss_bufs[b], sems[b]).wait()

    def process(b):
        ids_v = ids_bufs[b]
        loss_v = loss_bufs[b]

        @pl.loop(0, CHUNK // 64)
        def _vec(j):
            for u in range(4):
                o = (j * 4 + u) * 16
                idx = ids_v[pl.ds(o, 16)]
                x = loss_v[pl.ds(o, 16)]
                plsc.addupdate_scatter(cnt_h, [idx], ones16)
                plsc.addupdate_scatter(sum_h, [idx], x)

    issue(0, 0)
    issue(1, 1)

    @pl.loop(0, H // 16)
    def _zero(i):
        cnt_h[pl.ds(i * 16, 16)] = zeros16
        sum_h[pl.ds(i * 16, 16)] = zeros16

    @pl.loop(0, NCHUNK - 2, step=2)
    def _outer(ci):
        for b in range(2):
            drain(ci + b, b)
            process(b)
            issue(ci + b + 2, b)

    drain(NCHUNK - 2, 0)
    process(0)
    drain(NCHUNK - 1, 1)
    process(1)

    pltpu.sync_copy(cnt_h, cnt_out.at[wid])
    pltpu.sync_copy(sum_h, sum_out.at[wid])


def _masked_roll_add(x, sh, axis, pos):
    return x + jnp.where(pos >= sh, pltpu.roll(x, sh, axis), 0.0)


def _select_body(cnt_ref, sum_ref, out_ref, acc_c, acc_s):
    step = pl.program_id(0)

    @pl.when(step == 0)
    def _init():
        acc_c[...] = jnp.zeros_like(acc_c)
        acc_s[...] = jnp.zeros_like(acc_s)

    acc_c[...] += cnt_ref[0]
    acc_s[...] += sum_ref[0]

    @pl.when(step == NTILES - 1)
    def _finish():
        lane = lax.broadcasted_iota(jnp.int32, (HROWS, HCOLS), 1)
        row = lax.broadcasted_iota(jnp.int32, (HROWS, HCOLS), 0)
        valid = (row * HCOLS + lane) < NBUCKETS
        cnt = jnp.where(valid, acc_c[...], 0.0)
        lsum = jnp.where(valid, acc_s[...], 0.0)

        x = cnt
        for sh in (1, 2, 4, 8, 16, 32, 64):
            x = _masked_roll_add(x, sh, 1, lane)
        rowt = jnp.broadcast_to(x[:, HCOLS - 1:HCOLS], (HROWS, HCOLS))
        z = rowt
        for sh in (1, 2, 4, 8, 16, 32, 64, 128, 256):
            z = _masked_roll_add(z, sh, 0, row)
        excl = (x + (z - rowt)) - cnt

        total = jnp.sum(cnt)
        k = jnp.minimum(jnp.float32(K_KEPT), total)
        take = jnp.clip(k - excl, 0.0, cnt)
        num = jnp.sum(lsum * take / jnp.maximum(cnt, 1.0))
        out_ref[...] = jnp.reshape(num / jnp.maximum(k, 1.0), (1, 1))


def _make_elemwise():
    return pl.pallas_call(
        _elemwise_body,
        grid=(ROWS // BLK_ROWS,),
        in_specs=[
            pl.BlockSpec((BLK_ROWS, COLS), lambda i: (i, 0)),
            pl.BlockSpec((BLK_ROWS, COLS), lambda i: (i, 0)),
        ],
        out_specs=[
            pl.BlockSpec((BLK_ROWS, COLS), lambda i: (i, 0)),
            pl.BlockSpec((BLK_ROWS, COLS), lambda i: (i, 0)),
        ],
        out_shape=[
            jax.ShapeDtypeStruct((ROWS, COLS), jnp.int32),
            jax.ShapeDtypeStruct((ROWS, COLS), jnp.float32),
        ],
    )


def _make_hist():
    mesh = plsc.VectorSubcoreMesh(core_axis_name="c", subcore_axis_name="s")
    return pl.kernel(
        _hist_body,
        out_type=(
            jax.ShapeDtypeStruct((NTILES, H), jnp.float32),
            jax.ShapeDtypeStruct((NTILES, H), jnp.float32),
        ),
        mesh=mesh,
        scratch_types=[
            pltpu.VMEM((CHUNK,), jnp.int32),
            pltpu.VMEM((CHUNK,), jnp.float32),
            pltpu.VMEM((CHUNK,), jnp.int32),
            pltpu.VMEM((CHUNK,), jnp.float32),
            pltpu.VMEM((H,), jnp.float32),
            pltpu.VMEM((H,), jnp.float32),
            pltpu.SemaphoreType.DMA,
            pltpu.SemaphoreType.DMA,
        ],
        compiler_params=pltpu.CompilerParams(needs_layout_passes=False),
    )


def _make_select():
    return pl.pallas_call(
        _select_body,
        grid=(NTILES,),
        in_specs=[
            pl.BlockSpec((1, HROWS, HCOLS), lambda i: (i, 0, 0)),
            pl.BlockSpec((1, HROWS, HCOLS), lambda i: (i, 0, 0)),
        ],
        out_specs=pl.BlockSpec((1, 1), lambda i: (0, 0)),
        out_shape=jax.ShapeDtypeStruct((1, 1), jnp.float32),
        scratch_shapes=[
            pltpu.VMEM((HROWS, HCOLS), jnp.float32),
            pltpu.VMEM((HROWS, HCOLS), jnp.float32),
        ],
    )


def kernel(score, target):
    s = score.reshape(ROWS, COLS)
    t = target.reshape(ROWS, COLS)
    ids, losses = _make_elemwise()(s, t)
    cnt_h, sum_h = _make_hist()(ids.reshape(N), losses.reshape(N))
    out = _make_select()(
        cnt_h.reshape(NTILES, HROWS, HCOLS),
        sum_h.reshape(NTILES, HROWS, HCOLS),
    )
    return out.reshape(())


# speedup vs baseline: 21.1360x; 1.2481x over previous
import functools

import jax
import jax.numpy as jnp
from jax import lax
from jax.experimental import pallas as pl
from jax.experimental.pallas import tpu as pltpu
from jax.experimental.pallas import tpu_sc as plsc

THRESH = 0.2
MIN_KEPT_FRAC = 0.01
ABS_THRESH = 0.8472978603872037

N = 8 * 512 * 512
OUT_ROWS = N // 128

HROWS, HCOLS = 264, 128
H = HROWS * HCOLS
NBUCKETS = 32768
TRASH = NBUCKETS

NTILES = 32
PER_TILE = N // NTILES
CHUNK = 4096

K_KEPT = int(MIN_KEPT_FRAC * N)


_SOFTPLUS_COEFFS = (
    0.6931471710278221,
    -0.4999994142888598,
    0.12499137715952302,
    5.171521911596488e-05,
    -0.005358852851406005,
    0.0002188750327285275,
    0.000211068980896383,
)


def _elemwise_body(s_ref, t_ref, id_ref, loss_ref):
    s = s_ref[0]
    t = t_ref[0]
    a = jnp.abs(s)
    sp = jnp.float32(_SOFTPLUS_COEFFS[-1])
    for coef in _SOFTPLUS_COEFFS[-2::-1]:
        sp = sp * a + jnp.float32(coef)
    loss = jnp.maximum(s, 0.0) - s * t + sp
    mask = a < ABS_THRESH
    bits = lax.bitcast_convert_type(a, jnp.int32)
    bucket = jnp.minimum(lax.shift_right_logical(bits, 15), NBUCKETS - 1)
    id_ref[...] = jnp.where(mask, bucket, TRASH)
    loss_ref[...] = loss


NCHUNK = PER_TILE // CHUNK


def _hist_body(ids_hbm, loss_hbm, cnt_out, sum_out,
               ids_v0, loss_v0, ids_v1, loss_v1, cnt_h, sum_h, sem0, sem1):
    c = lax.axis_index("c")
    s = lax.axis_index("s")
    wid = s * 2 + c
    base = wid * PER_TILE

    zeros16 = jnp.zeros((16,), jnp.float32)
    ones16 = jnp.ones((16,), jnp.float32)

    ids_bufs = (ids_v0, ids_v1)
    loss_bufs = (loss_v0, loss_v1)
    sems = (sem0, sem1)

    def issue(ci, b):
        off = base + ci * CHUNK
        pltpu.async_copy(ids_hbm.at[pl.ds(off, CHUNK)], ids_bufs[b], sems[b])
        pltpu.async_copy(loss_hbm.at[pl.ds(off, CHUNK)], loss_bufs[b], sems[b])

    def drain(ci, b):
        off = base + ci * CHUNK
        pltpu.make_async_copy(ids_hbm.at[pl.ds(off, CHUNK)], ids_bufs[b], sems[b]).wait()
        pltpu.make_async_copy(loss_hbm.at[pl.ds(off, CHUNK)], loss_bufs[b], sems[b]).wait()

    def process(b):
        ids_v = ids_bufs[b]
        loss_v = loss_bufs[b]

        @pl.loop(0, CHUNK // 64)
        def _vec(j):
            for u in range(4):
                o = (j * 4 + u) * 16
                idx = ids_v[pl.ds(o, 16)]
                x = loss_v[pl.ds(o, 16)]
                plsc.addupdate_scatter(cnt_h, [idx], ones16)
                plsc.addupdate_scatter(sum_h, [idx], x)

    issue(0, 0)
    issue(1, 1)

    @pl.loop(0, H // 16)
    def _zero(i):
        cnt_h[pl.ds(i * 16, 16)] = zeros16
        sum_h[pl.ds(i * 16, 16)] = zeros16

    @pl.loop(0, NCHUNK - 2, step=2)
    def _outer(ci):
        for b in range(2):
            drain(ci + b, b)
            process(b)
            issue(ci + b + 2, b)

    drain(NCHUNK - 2, 0)
    process(0)
    drain(NCHUNK - 1, 1)
    process(1)

    pltpu.sync_copy(cnt_h, cnt_out.at[pl.ds(wid * H, H)])
    pltpu.sync_copy(sum_h, sum_out.at[pl.ds(wid * H, H)])


def _masked_roll_add(x, sh, axis, pos):
    return x + jnp.where(pos >= sh, pltpu.roll(x, sh, axis), 0.0)


def _select_body(cnt_ref, sum_ref, out_ref, acc_c, acc_s):
    step = pl.program_id(0)

    @pl.when(step == 0)
    def _init():
        acc_c[...] = jnp.zeros_like(acc_c)
        acc_s[...] = jnp.zeros_like(acc_s)

    acc_c[...] += cnt_ref[...]
    acc_s[...] += sum_ref[...]

    @pl.when(step == NTILES - 1)
    def _finish():
        lane = lax.broadcasted_iota(jnp.int32, (HROWS, HCOLS), 1)
        row = lax.broadcasted_iota(jnp.int32, (HROWS, HCOLS), 0)
        valid = (row * HCOLS + lane) < NBUCKETS
        cnt = jnp.where(valid, acc_c[...], 0.0)
        lsum = jnp.where(valid, acc_s[...], 0.0)

        x = cnt
        for sh in (1, 2, 4, 8, 16, 32, 64):
            x = _masked_roll_add(x, sh, 1, lane)
        rowt = jnp.broadcast_to(x[:, HCOLS - 1:HCOLS], (HROWS, HCOLS))
        z = rowt
        for sh in (1, 2, 4, 8, 16, 32, 64, 128, 256):
            z = _masked_roll_add(z, sh, 0, row)
        excl = (x + (z - rowt)) - cnt

        total = jnp.sum(cnt)
        k = jnp.minimum(jnp.float32(K_KEPT), total)
        take = jnp.clip(k - excl, 0.0, cnt)
        num = jnp.sum(lsum * take / jnp.maximum(cnt, 1.0))
        out_ref[...] = jnp.reshape(num / jnp.maximum(k, 1.0), (1, 1))


def _make_elemwise():
    return pl.pallas_call(
        _elemwise_body,
        grid=(32,),
        in_specs=[
            pl.BlockSpec((1, 512, 128), lambda i: (i // 4, 0, i % 4)),
            pl.BlockSpec((1, 512, 128), lambda i: (i // 4, 0, i % 4)),
        ],
        out_specs=[
            pl.BlockSpec((512, 128), lambda i: (i, 0)),
            pl.BlockSpec((512, 128), lambda i: (i, 0)),
        ],
        out_shape=[
            jax.ShapeDtypeStruct((OUT_ROWS, 128), jnp.int32),
            jax.ShapeDtypeStruct((OUT_ROWS, 128), jnp.float32),
        ],
    )


def _make_hist():
    mesh = plsc.VectorSubcoreMesh(core_axis_name="c", subcore_axis_name="s")
    return pl.kernel(
        _hist_body,
        out_type=(
            jax.ShapeDtypeStruct((NTILES * H,), jnp.float32),
            jax.ShapeDtypeStruct((NTILES * H,), jnp.float32),
        ),
        mesh=mesh,
        scratch_types=[
            pltpu.VMEM((CHUNK,), jnp.int32),
            pltpu.VMEM((CHUNK,), jnp.float32),
            pltpu.VMEM((CHUNK,), jnp.int32),
            pltpu.VMEM((CHUNK,), jnp.float32),
            pltpu.VMEM((H,), jnp.float32),
            pltpu.VMEM((H,), jnp.float32),
            pltpu.SemaphoreType.DMA,
            pltpu.SemaphoreType.DMA,
        ],
        compiler_params=pltpu.CompilerParams(needs_layout_passes=False),
    )


def _make_select():
    return pl.pallas_call(
        _select_body,
        grid=(NTILES,),
        in_specs=[
            pl.BlockSpec((HROWS, HCOLS), lambda i: (i, 0)),
            pl.BlockSpec((HROWS, HCOLS), lambda i: (i, 0)),
        ],
        out_specs=pl.BlockSpec((1, 1), lambda i: (0, 0)),
        out_shape=jax.ShapeDtypeStruct((1, 1), jnp.float32),
        scratch_shapes=[
            pltpu.VMEM((HROWS, HCOLS), jnp.float32),
            pltpu.VMEM((HROWS, HCOLS), jnp.float32),
        ],
    )


def kernel(score, target):
    ids, losses = _make_elemwise()(score, target)
    cnt_h, sum_h = _make_hist()(ids.reshape(N), losses.reshape(N))
    out = _make_select()(
        cnt_h.reshape(NTILES * HROWS, HCOLS),
        sum_h.reshape(NTILES * HROWS, HCOLS),
    )
    return out.reshape(())


# speedup vs baseline: 26.4419x; 1.2510x over previous
import functools

import jax
import jax.numpy as jnp
from jax import lax
from jax.experimental import pallas as pl
from jax.experimental.pallas import tpu as pltpu
from jax.experimental.pallas import tpu_sc as plsc

THRESH = 0.2
MIN_KEPT_FRAC = 0.01
ABS_THRESH = 0.8472978603872037

N = 8 * 512 * 512
OUT_ROWS = N // 128

HROWS, HCOLS = 264, 128
H = HROWS * HCOLS
NBUCKETS = 32768
TRASH = NBUCKETS

NTILES = 32
PER_TILE = N // NTILES
CHUNK = 4096

K_KEPT = int(MIN_KEPT_FRAC * N)

PACK = 4096.0


_SOFTPLUS_COEFFS = (
    0.6931471710278221,
    -0.4999994142888598,
    0.12499137715952302,
    5.171521911596488e-05,
    -0.005358852851406005,
    0.0002188750327285275,
    0.000211068980896383,
)


def _elemwise_body(s_ref, t_ref, id_ref, loss_ref):
    s = s_ref[0]
    t = t_ref[0]
    a = jnp.abs(s)
    sp = jnp.float32(_SOFTPLUS_COEFFS[-1])
    for coef in _SOFTPLUS_COEFFS[-2::-1]:
        sp = sp * a + jnp.float32(coef)
    loss = jnp.maximum(s, 0.0) - s * t + sp
    mask = a < ABS_THRESH
    bits = lax.bitcast_convert_type(a, jnp.int32)
    bucket = jnp.minimum(lax.shift_right_logical(bits, 15), NBUCKETS - 1)
    id_ref[...] = jnp.where(mask, bucket, TRASH)
    loss_ref[...] = jnp.where(mask, loss + PACK, 0.0)


NCHUNK = PER_TILE // CHUNK


def _hist_body(ids_hbm, val_hbm, hist_out,
               ids_v0, val_v0, ids_v1, val_v1, hist_h, sem0, sem1):
    c = lax.axis_index("c")
    s = lax.axis_index("s")
    wid = s * 2 + c
    base = wid * PER_TILE

    zeros16 = jnp.zeros((16,), jnp.float32)

    ids_bufs = (ids_v0, ids_v1)
    val_bufs = (val_v0, val_v1)
    sems = (sem0, sem1)

    def issue(ci, b):
        off = base + ci * CHUNK
        pltpu.async_copy(ids_hbm.at[pl.ds(off, CHUNK)], ids_bufs[b], sems[b])
        pltpu.async_copy(val_hbm.at[pl.ds(off, CHUNK)], val_bufs[b], sems[b])

    def drain(ci, b):
        off = base + ci * CHUNK
        pltpu.make_async_copy(ids_hbm.at[pl.ds(off, CHUNK)], ids_bufs[b], sems[b]).wait()
        pltpu.make_async_copy(val_hbm.at[pl.ds(off, CHUNK)], val_bufs[b], sems[b]).wait()

    def process(b):
        ids_v = ids_bufs[b]
        val_v = val_bufs[b]

        @pl.loop(0, CHUNK // 64)
        def _vec(j):
            for u in range(4):
                o = (j * 4 + u) * 16
                idx = ids_v[pl.ds(o, 16)]
                x = val_v[pl.ds(o, 16)]
                plsc.addupdate_scatter(hist_h, [idx], x)

    issue(0, 0)
    issue(1, 1)

    @pl.loop(0, H // 16)
    def _zero(i):
        hist_h[pl.ds(i * 16, 16)] = zeros16

    @pl.loop(0, NCHUNK - 2, step=2)
    def _outer(ci):
        for b in range(2):
            drain(ci + b, b)
            process(b)
            issue(ci + b + 2, b)

    drain(NCHUNK - 2, 0)
    process(0)
    drain(NCHUNK - 1, 1)
    process(1)

    pltpu.sync_copy(hist_h, hist_out.at[pl.ds(wid * H, H)])


def _masked_roll_add(x, sh, axis, pos):
    return x + jnp.where(pos >= sh, pltpu.roll(x, sh, axis), 0.0)


def _select_body(hist_ref, out_ref, acc_c, acc_s):
    step = pl.program_id(0)

    @pl.when(step == 0)
    def _init():
        acc_c[...] = jnp.zeros_like(acc_c)
        acc_s[...] = jnp.zeros_like(acc_s)

    x = hist_ref[...]
    c = jnp.floor(x * (1.0 / PACK))
    acc_c[...] += c
    acc_s[...] += x - PACK * c

    @pl.when(step == NTILES - 1)
    def _finish():
        lane = lax.broadcasted_iota(jnp.int32, (HROWS, HCOLS), 1)
        row = lax.broadcasted_iota(jnp.int32, (HROWS, HCOLS), 0)
        valid = (row * HCOLS + lane) < NBUCKETS
        cnt = jnp.where(valid, acc_c[...], 0.0)
        lsum = jnp.where(valid, acc_s[...], 0.0)

        x = cnt
        for sh in (1, 2, 4, 8, 16, 32, 64):
            x = _masked_roll_add(x, sh, 1, lane)
        rowt = jnp.broadcast_to(x[:, HCOLS - 1:HCOLS], (HROWS, HCOLS))
        z = rowt
        for sh in (1, 2, 4, 8, 16, 32, 64, 128, 256):
            z = _masked_roll_add(z, sh, 0, row)
        excl = (x + (z - rowt)) - cnt

        total = jnp.sum(cnt)
        k = jnp.minimum(jnp.float32(K_KEPT), total)
        take = jnp.clip(k - excl, 0.0, cnt)
        num = jnp.sum(lsum * take / jnp.maximum(cnt, 1.0))
        out_ref[...] = jnp.reshape(num / jnp.maximum(k, 1.0), (1, 1))


def _make_elemwise():
    return pl.pallas_call(
        _elemwise_body,
        grid=(32,),
        in_specs=[
            pl.BlockSpec((1, 512, 128), lambda i: (i // 4, 0, i % 4)),
            pl.BlockSpec((1, 512, 128), lambda i: (i // 4, 0, i % 4)),
        ],
        out_specs=[
            pl.BlockSpec((512, 128), lambda i: (i, 0)),
            pl.BlockSpec((512, 128), lambda i: (i, 0)),
        ],
        out_shape=[
            jax.ShapeDtypeStruct((OUT_ROWS, 128), jnp.int32),
            jax.ShapeDtypeStruct((OUT_ROWS, 128), jnp.float32),
        ],
    )


def _make_hist():
    mesh = plsc.VectorSubcoreMesh(core_axis_name="c", subcore_axis_name="s")
    return pl.kernel(
        _hist_body,
        out_type=jax.ShapeDtypeStruct((NTILES * H,), jnp.float32),
        mesh=mesh,
        scratch_types=[
            pltpu.VMEM((CHUNK,), jnp.int32),
            pltpu.VMEM((CHUNK,), jnp.float32),
            pltpu.VMEM((CHUNK,), jnp.int32),
            pltpu.VMEM((CHUNK,), jnp.float32),
            pltpu.VMEM((H,), jnp.float32),
            pltpu.SemaphoreType.DMA,
            pltpu.SemaphoreType.DMA,
        ],
        compiler_params=pltpu.CompilerParams(needs_layout_passes=False),
    )


def _make_select():
    return pl.pallas_call(
        _select_body,
        grid=(NTILES,),
        in_specs=[
            pl.BlockSpec((HROWS, HCOLS), lambda i: (i, 0)),
        ],
        out_specs=pl.BlockSpec((1, 1), lambda i: (0, 0)),
        out_shape=jax.ShapeDtypeStruct((1, 1), jnp.float32),
        scratch_shapes=[
            pltpu.VMEM((HROWS, HCOLS), jnp.float32),
            pltpu.VMEM((HROWS, HCOLS), jnp.float32),
        ],
    )


def kernel(score, target):
    ids, vals = _make_elemwise()(score, target)
    hist = _make_hist()(ids.reshape(N), vals.reshape(N))
    out = _make_select()(hist.reshape(NTILES * HROWS, HCOLS))
    return out.reshape(())


# speedup vs baseline: 28.8292x; 1.0903x over previous
import functools

import jax
import jax.numpy as jnp
from jax import lax
from jax.experimental import pallas as pl
from jax.experimental.pallas import tpu as pltpu
from jax.experimental.pallas import tpu_sc as plsc

THRESH = 0.2
MIN_KEPT_FRAC = 0.01
ABS_THRESH = 0.8472978603872037

N = 8 * 512 * 512
OUT_ROWS = N // 128

HROWS, HCOLS = 72, 128
H = HROWS * HCOLS
NBUCKETS = 8192
BUCKET_SHIFT = 17
TRASH = NBUCKETS

NTILES = 32
PER_TILE = N // NTILES
CHUNK = 4096

K_KEPT = int(MIN_KEPT_FRAC * N)

PACK = 4096.0


_SOFTPLUS_COEFFS = (
    0.6931471710278221,
    -0.4999994142888598,
    0.12499137715952302,
    5.171521911596488e-05,
    -0.005358852851406005,
    0.0002188750327285275,
    0.000211068980896383,
)


def _elemwise_body(s_ref, t_ref, id_ref, loss_ref):
    s = s_ref[0]
    t = t_ref[0]
    a = jnp.abs(s)
    sp = jnp.float32(_SOFTPLUS_COEFFS[-1])
    for coef in _SOFTPLUS_COEFFS[-2::-1]:
        sp = sp * a + jnp.float32(coef)
    loss = jnp.maximum(s, 0.0) - s * t + sp
    mask = a < ABS_THRESH
    bits = lax.bitcast_convert_type(a, jnp.int32)
    bucket = jnp.minimum(lax.shift_right_logical(bits, BUCKET_SHIFT), NBUCKETS - 1)
    id_ref[...] = jnp.where(mask, bucket, TRASH)
    loss_ref[...] = jnp.where(mask, loss + PACK, 0.0)


NCHUNK = PER_TILE // CHUNK


def _hist_body(ids_hbm, val_hbm, hist_out,
               ids_v0, val_v0, ids_v1, val_v1, hist_h, sem0, sem1):
    c = lax.axis_index("c")
    s = lax.axis_index("s")
    wid = s * 2 + c
    base = wid * PER_TILE

    zeros16 = jnp.zeros((16,), jnp.float32)

    ids_bufs = (ids_v0, ids_v1)
    val_bufs = (val_v0, val_v1)
    sems = (sem0, sem1)

    def issue(ci, b):
        off = base + ci * CHUNK
        pltpu.async_copy(ids_hbm.at[pl.ds(off, CHUNK)], ids_bufs[b], sems[b])
        pltpu.async_copy(val_hbm.at[pl.ds(off, CHUNK)], val_bufs[b], sems[b])

    def drain(ci, b):
        off = base + ci * CHUNK
        pltpu.make_async_copy(ids_hbm.at[pl.ds(off, CHUNK)], ids_bufs[b], sems[b]).wait()
        pltpu.make_async_copy(val_hbm.at[pl.ds(off, CHUNK)], val_bufs[b], sems[b]).wait()

    def process(b):
        ids_v = ids_bufs[b]
        val_v = val_bufs[b]

        @pl.loop(0, CHUNK // 64)
        def _vec(j):
            for u in range(4):
                o = (j * 4 + u) * 16
                idx = ids_v[pl.ds(o, 16)]
                x = val_v[pl.ds(o, 16)]
                plsc.addupdate_scatter(hist_h, [idx], x)

    issue(0, 0)
    issue(1, 1)

    @pl.loop(0, H // 16)
    def _zero(i):
        hist_h[pl.ds(i * 16, 16)] = zeros16

    @pl.loop(0, NCHUNK - 2, step=2)
    def _outer(ci):
        for b in range(2):
            drain(ci + b, b)
            process(b)
            issue(ci + b + 2, b)

    drain(NCHUNK - 2, 0)
    process(0)
    drain(NCHUNK - 1, 1)
    process(1)

    pltpu.sync_copy(hist_h, hist_out.at[pl.ds(wid * H, H)])


def _masked_roll_add(x, sh, axis, pos):
    return x + jnp.where(pos >= sh, pltpu.roll(x, sh, axis), 0.0)


def _select_body(hist_ref, out_ref, acc_c, acc_s):
    step = pl.program_id(0)

    @pl.when(step == 0)
    def _init():
        acc_c[...] = jnp.zeros_like(acc_c)
        acc_s[...] = jnp.zeros_like(acc_s)

    x = hist_ref[...]
    c = jnp.floor(x * (1.0 / PACK))
    acc_c[...] += c
    acc_s[...] += x - PACK * c

    @pl.when(step == NTILES - 1)
    def _finish():
        lane = lax.broadcasted_iota(jnp.int32, (HROWS, HCOLS), 1)
        row = lax.broadcasted_iota(jnp.int32, (HROWS, HCOLS), 0)
        valid = (row * HCOLS + lane) < NBUCKETS
        cnt = jnp.where(valid, acc_c[...], 0.0)
        lsum = jnp.where(valid, acc_s[...], 0.0)

        x = cnt
        for sh in (1, 2, 4, 8, 16, 32, 64):
            x = _masked_roll_add(x, sh, 1, lane)
        rowt = jnp.broadcast_to(x[:, HCOLS - 1:HCOLS], (HROWS, HCOLS))
        z = rowt
        for sh in (1, 2, 4, 8, 16, 32, 64):
            z = _masked_roll_add(z, sh, 0, row)
        excl = (x + (z - rowt)) - cnt

        total = jnp.sum(cnt)
        k = jnp.minimum(jnp.float32(K_KEPT), total)
        take = jnp.clip(k - excl, 0.0, cnt)
        num = jnp.sum(lsum * take / jnp.maximum(cnt, 1.0))
        out_ref[...] = jnp.reshape(num / jnp.maximum(k, 1.0), (1, 1))


def _make_elemwise():
    return pl.pallas_call(
        _elemwise_body,
        grid=(32,),
        in_specs=[
            pl.BlockSpec((1, 512, 128), lambda i: (i // 4, 0, i % 4)),
            pl.BlockSpec((1, 512, 128), lambda i: (i // 4, 0, i % 4)),
        ],
        out_specs=[
            pl.BlockSpec((512, 128), lambda i: (i, 0)),
            pl.BlockSpec((512, 128), lambda i: (i, 0)),
        ],
        out_shape=[
            jax.ShapeDtypeStruct((OUT_ROWS, 128), jnp.int32),
            jax.ShapeDtypeStruct((OUT_ROWS, 128), jnp.float32),
        ],
    )


def _make_hist():
    mesh = plsc.VectorSubcoreMesh(core_axis_name="c", subcore_axis_name="s")
    return pl.kernel(
        _hist_body,
        out_type=jax.ShapeDtypeStruct((NTILES * H,), jnp.float32),
        mesh=mesh,
        scratch_types=[
            pltpu.VMEM((CHUNK,), jnp.int32),
            pltpu.VMEM((CHUNK,), jnp.float32),
            pltpu.VMEM((CHUNK,), jnp.int32),
            pltpu.VMEM((CHUNK,), jnp.float32),
            pltpu.VMEM((H,), jnp.float32),
            pltpu.SemaphoreType.DMA,
            pltpu.SemaphoreType.DMA,
        ],
        compiler_params=pltpu.CompilerParams(needs_layout_passes=False),
    )


def _make_select():
    return pl.pallas_call(
        _select_body,
        grid=(NTILES,),
        in_specs=[
            pl.BlockSpec((HROWS, HCOLS), lambda i: (i, 0)),
        ],
        out_specs=pl.BlockSpec((1, 1), lambda i: (0, 0)),
        out_shape=jax.ShapeDtypeStruct((1, 1), jnp.float32),
        scratch_shapes=[
            pltpu.VMEM((HROWS, HCOLS), jnp.float32),
            pltpu.VMEM((HROWS, HCOLS), jnp.float32),
        ],
    )


def kernel(score, target):
    ids, vals = _make_elemwise()(score, target)
    hist = _make_hist()(ids.reshape(N), vals.reshape(N))
    out = _make_select()(hist.reshape(NTILES * HROWS, HCOLS))
    return out.reshape(())


# speedup vs baseline: 30.5251x; 1.0588x over previous
import functools

import jax
import jax.numpy as jnp
from jax import lax
from jax.experimental import pallas as pl
from jax.experimental.pallas import tpu as pltpu
from jax.experimental.pallas import tpu_sc as plsc

THRESH = 0.2
MIN_KEPT_FRAC = 0.01
ABS_THRESH = 0.8472978603872037

N = 8 * 512 * 512
OUT_ROWS = N // 128

HROWS, HCOLS = 72, 128
H = HROWS * HCOLS
NBUCKETS = 8192
BUCKET_SHIFT = 17
TRASH = NBUCKETS

NTILES = 32
PER_TILE = N // NTILES
CHUNK = 8192

K_KEPT = int(MIN_KEPT_FRAC * N)

PACK = 4096.0


_SOFTPLUS_COEFFS = (
    0.6931471710278221,
    -0.4999994142888598,
    0.12499137715952302,
    5.171521911596488e-05,
    -0.005358852851406005,
    0.0002188750327285275,
    0.000211068980896383,
)


def _elemwise_body(s_ref, t_ref, id_ref):
    s = s_ref[0]
    t = t_ref[0]
    a = jnp.abs(s)
    sp = jnp.float32(_SOFTPLUS_COEFFS[-1])
    for coef in _SOFTPLUS_COEFFS[-2::-1]:
        sp = sp * a + jnp.float32(coef)
    loss = jnp.maximum(s, 0.0) - s * t + sp
    mask = a < ABS_THRESH
    bits = lax.bitcast_convert_type(a, jnp.int32)
    bucket = jnp.minimum(lax.shift_right_logical(bits, BUCKET_SHIFT), NBUCKETS - 1)
    lq = (loss * 65536.0 + 0.5).astype(jnp.int32)
    word = lax.shift_left(jnp.where(mask, bucket, TRASH), 17)
    id_ref[...] = word | jnp.where(mask, lq, 0)


NCHUNK = PER_TILE // CHUNK


def _hist_body(word_hbm, hist_out, w_v0, w_v1, hist_h, sem0, sem1):
    c = lax.axis_index("c")
    s = lax.axis_index("s")
    wid = s * 2 + c
    base = wid * PER_TILE

    zeros16 = jnp.zeros((16,), jnp.float32)

    word_bufs = (w_v0, w_v1)
    sems = (sem0, sem1)

    def issue(ci, b):
        off = base + ci * CHUNK
        pltpu.async_copy(word_hbm.at[pl.ds(off, CHUNK)], word_bufs[b], sems[b])

    def drain(ci, b):
        off = base + ci * CHUNK
        pltpu.make_async_copy(word_hbm.at[pl.ds(off, CHUNK)], word_bufs[b], sems[b]).wait()

    def process(b):
        w_v = word_bufs[b]

        @pl.loop(0, CHUNK // 64)
        def _vec(j):
            for u in range(4):
                o = (j * 4 + u) * 16
                w = w_v[pl.ds(o, 16)]
                idx = lax.shift_right_logical(w, 17)
                lq = w & 0x1FFFF
                v = lq.astype(jnp.float32) * (1.0 / 65536.0) + PACK
                plsc.addupdate_scatter(hist_h, [idx], v)

    issue(0, 0)
    issue(1, 1)

    @pl.loop(0, H // 16)
    def _zero(i):
        hist_h[pl.ds(i * 16, 16)] = zeros16

    @pl.loop(0, NCHUNK - 2, step=2)
    def _outer(ci):
        for b in range(2):
            drain(ci + b, b)
            process(b)
            issue(ci + b + 2, b)

    drain(NCHUNK - 2, 0)
    process(0)
    drain(NCHUNK - 1, 1)
    process(1)

    pltpu.sync_copy(hist_h, hist_out.at[pl.ds(wid * H, H)])


def _masked_roll_add(x, sh, axis, pos):
    return x + jnp.where(pos >= sh, pltpu.roll(x, sh, axis), 0.0)


def _select_body(hist_ref, out_ref):
    acc_c = jnp.zeros((HROWS, HCOLS), jnp.float32)
    acc_s = jnp.zeros((HROWS, HCOLS), jnp.float32)
    for w in range(NTILES):
        x = hist_ref[pl.ds(w * HROWS, HROWS), :]
        c = jnp.floor(x * (1.0 / PACK))
        acc_c += c
        acc_s += x - PACK * c

    lane = lax.broadcasted_iota(jnp.int32, (HROWS, HCOLS), 1)
    row = lax.broadcasted_iota(jnp.int32, (HROWS, HCOLS), 0)
    valid = (row * HCOLS + lane) < NBUCKETS
    cnt = jnp.where(valid, acc_c, 0.0)
    lsum = jnp.where(valid, acc_s, 0.0)

    x = cnt
    for sh in (1, 2, 4, 8, 16, 32, 64):
        x = _masked_roll_add(x, sh, 1, lane)
    rowt = jnp.broadcast_to(x[:, HCOLS - 1:HCOLS], (HROWS, HCOLS))
    z = rowt
    for sh in (1, 2, 4, 8, 16, 32, 64):
        z = _masked_roll_add(z, sh, 0, row)
    excl = (x + (z - rowt)) - cnt

    total = jnp.sum(cnt)
    k = jnp.minimum(jnp.float32(K_KEPT), total)
    take = jnp.clip(k - excl, 0.0, cnt)
    num = jnp.sum(lsum * take / jnp.maximum(cnt, 1.0))
    out_ref[...] = jnp.reshape(num / jnp.maximum(k, 1.0), (1, 1))


def _make_elemwise():
    return pl.pallas_call(
        _elemwise_body,
        grid=(32,),
        in_specs=[
            pl.BlockSpec((1, 512, 128), lambda i: (i // 4, 0, i % 4)),
            pl.BlockSpec((1, 512, 128), lambda i: (i // 4, 0, i % 4)),
        ],
        out_specs=pl.BlockSpec((512, 128), lambda i: (i, 0)),
        out_shape=jax.ShapeDtypeStruct((OUT_ROWS, 128), jnp.int32),
    )


def _make_hist():
    mesh = plsc.VectorSubcoreMesh(core_axis_name="c", subcore_axis_name="s")
    return pl.kernel(
        _hist_body,
        out_type=jax.ShapeDtypeStruct((NTILES * H,), jnp.float32),
        mesh=mesh,
        scratch_types=[
            pltpu.VMEM((CHUNK,), jnp.int32),
            pltpu.VMEM((CHUNK,), jnp.int32),
            pltpu.VMEM((H,), jnp.float32),
            pltpu.SemaphoreType.DMA,
            pltpu.SemaphoreType.DMA,
        ],
        compiler_params=pltpu.CompilerParams(needs_layout_passes=False),
    )


def _make_select():
    return pl.pallas_call(
        _select_body,
        out_shape=jax.ShapeDtypeStruct((1, 1), jnp.float32),
    )


def kernel(score, target):
    words = _make_elemwise()(score, target)
    hist = _make_hist()(words.reshape(N))
    out = _make_select()(hist.reshape(NTILES * HROWS, HCOLS))
    return out.reshape(())


# speedup vs baseline: 31.9014x; 1.0451x over previous
import functools

import jax
import jax.numpy as jnp
from jax import lax
from jax.experimental import pallas as pl
from jax.experimental.pallas import tpu as pltpu
from jax.experimental.pallas import tpu_sc as plsc

THRESH = 0.2
MIN_KEPT_FRAC = 0.01
ABS_THRESH = 0.8472978603872037

N = 8 * 512 * 512
OUT_ROWS = N // 128

HROWS, HCOLS = 72, 128
H = HROWS * HCOLS
NBUCKETS = 8192
BUCKET_SHIFT = 17
TRASH = NBUCKETS

NTILES = 32
HALF_N = N // 2
PER_TILE = HALF_N // NTILES
CHUNK = 8192

K_KEPT = int(MIN_KEPT_FRAC * N)

PACK = 4096.0


_SOFTPLUS_COEFFS = (
    0.6931471710278221,
    -0.4999994142888598,
    0.12499137715952302,
    5.171521911596488e-05,
    -0.005358852851406005,
    0.0002188750327285275,
    0.000211068980896383,
)


def _elemwise_body(s_ref, t_ref, id_ref):
    s = s_ref[0]
    t = t_ref[0]
    a = jnp.abs(s)
    sp = jnp.float32(_SOFTPLUS_COEFFS[-1])
    for coef in _SOFTPLUS_COEFFS[-2::-1]:
        sp = sp * a + jnp.float32(coef)
    loss = jnp.maximum(s, 0.0) - s * t + sp
    mask = a < ABS_THRESH
    bits = lax.bitcast_convert_type(a, jnp.int32)
    bucket = jnp.minimum(lax.shift_right_logical(bits, BUCKET_SHIFT), NBUCKETS - 1)
    lq = (loss * 65536.0 + 0.5).astype(jnp.int32)
    word = lax.shift_left(jnp.where(mask, bucket, TRASH), 17)
    id_ref[...] = word | jnp.where(mask, lq, 0)


NCHUNK = PER_TILE // CHUNK


def _hist_body(word_hbm, hist_out, w_v0, w_v1, hist_h, sem0, sem1):
    c = lax.axis_index("c")
    s = lax.axis_index("s")
    wid = s * 2 + c
    base = wid * PER_TILE

    zeros16 = jnp.zeros((16,), jnp.float32)

    word_bufs = (w_v0, w_v1)
    sems = (sem0, sem1)

    def issue(ci, b):
        off = base + ci * CHUNK
        pltpu.async_copy(word_hbm.at[pl.ds(off, CHUNK)], word_bufs[b], sems[b])

    def drain(ci, b):
        off = base + ci * CHUNK
        pltpu.make_async_copy(word_hbm.at[pl.ds(off, CHUNK)], word_bufs[b], sems[b]).wait()

    def process(b):
        w_v = word_bufs[b]

        @pl.loop(0, CHUNK // 64)
        def _vec(j):
            for u in range(4):
                o = (j * 4 + u) * 16
                w = w_v[pl.ds(o, 16)]
                idx = lax.shift_right_logical(w, 17)
                lq = w & 0x1FFFF
                v = lq.astype(jnp.float32) * (1.0 / 65536.0) + PACK
                plsc.addupdate_scatter(hist_h, [idx], v)

    issue(0, 0)
    issue(1, 1)

    @pl.loop(0, H // 16)
    def _zero(i):
        hist_h[pl.ds(i * 16, 16)] = zeros16

    @pl.loop(0, NCHUNK - 2, step=2)
    def _outer(ci):
        for b in range(2):
            drain(ci + b, b)
            process(b)
            issue(ci + b + 2, b)

    drain(NCHUNK - 2, 0)
    process(0)
    drain(NCHUNK - 1, 1)
    process(1)

    pltpu.sync_copy(hist_h, hist_out.at[pl.ds(wid * H, H)])


def _masked_roll_add(x, sh, axis, pos):
    return x + jnp.where(pos >= sh, pltpu.roll(x, sh, axis), 0.0)


def _select_body(hist0_ref, hist1_ref, out_ref):
    acc_c = jnp.zeros((HROWS, HCOLS), jnp.float32)
    acc_s = jnp.zeros((HROWS, HCOLS), jnp.float32)
    for hist_ref in (hist0_ref, hist1_ref):
        for w in range(NTILES):
            x = hist_ref[pl.ds(w * HROWS, HROWS), :]
            c = jnp.floor(x * (1.0 / PACK))
            acc_c += c
            acc_s += x - PACK * c

    lane = lax.broadcasted_iota(jnp.int32, (HROWS, HCOLS), 1)
    row = lax.broadcasted_iota(jnp.int32, (HROWS, HCOLS), 0)
    valid = (row * HCOLS + lane) < NBUCKETS
    cnt = jnp.where(valid, acc_c, 0.0)
    lsum = jnp.where(valid, acc_s, 0.0)

    x = cnt
    for sh in (1, 2, 4, 8, 16, 32, 64):
        x = _masked_roll_add(x, sh, 1, lane)
    rowt = jnp.broadcast_to(x[:, HCOLS - 1:HCOLS], (HROWS, HCOLS))
    z = rowt
    for sh in (1, 2, 4, 8, 16, 32, 64):
        z = _masked_roll_add(z, sh, 0, row)
    excl = (x + (z - rowt)) - cnt

    total = jnp.sum(cnt)
    k = jnp.minimum(jnp.float32(K_KEPT), total)
    take = jnp.clip(k - excl, 0.0, cnt)
    num = jnp.sum(lsum * take / jnp.maximum(cnt, 1.0))
    out_ref[...] = jnp.reshape(num / jnp.maximum(k, 1.0), (1, 1))


def _make_elemwise(img_off):
    return pl.pallas_call(
        _elemwise_body,
        grid=(16,),
        in_specs=[
            pl.BlockSpec((1, 512, 128), lambda i: (i // 4 + img_off, 0, i % 4)),
            pl.BlockSpec((1, 512, 128), lambda i: (i // 4 + img_off, 0, i % 4)),
        ],
        out_specs=pl.BlockSpec((512, 128), lambda i: (i, 0)),
        out_shape=jax.ShapeDtypeStruct((OUT_ROWS // 2, 128), jnp.int32),
    )


def _make_hist():
    mesh = plsc.VectorSubcoreMesh(core_axis_name="c", subcore_axis_name="s")
    return pl.kernel(
        _hist_body,
        out_type=jax.ShapeDtypeStruct((NTILES * H,), jnp.float32),
        mesh=mesh,
        scratch_types=[
            pltpu.VMEM((CHUNK,), jnp.int32),
            pltpu.VMEM((CHUNK,), jnp.int32),
            pltpu.VMEM((H,), jnp.float32),
            pltpu.SemaphoreType.DMA,
            pltpu.SemaphoreType.DMA,
        ],
        compiler_params=pltpu.CompilerParams(needs_layout_passes=False),
    )


def _make_select():
    return pl.pallas_call(
        _select_body,
        out_shape=jax.ShapeDtypeStruct((1, 1), jnp.float32),
    )


def kernel(score, target):
    elem0 = _make_elemwise(0)
    elem1 = _make_elemwise(4)
    hist_call = _make_hist()
    words0 = elem0(score, target)
    hist0 = hist_call(words0.reshape(HALF_N))
    words1 = elem1(score, target)
    hist1 = hist_call(words1.reshape(HALF_N))
    out = _make_select()(
        hist0.reshape(NTILES * HROWS, HCOLS),
        hist1.reshape(NTILES * HROWS, HCOLS),
    )
    return out.reshape(())
